# baseline (speedup 1.0000x reference)
import functools
import math

import jax
import jax.numpy as jnp
import numpy as np
from jax import lax
from jax.experimental import pallas as pl
from jax.experimental.pallas import tpu as pltpu
from jax.experimental.pallas import tpu_sc as plsc

_NC = 2
_NS = 16
_NW = _NC * _NS


def _sc_gather(table, idx_flat):
    B = idx_flat.shape[0]
    _, D = table.shape
    b_per_w = B // _NW
    CH = 16
    n_ch = b_per_w // CH
    mesh = plsc.VectorSubcoreMesh(core_axis_name="c", subcore_axis_name="s")

    NBUF = 6

    @functools.partial(
        pl.kernel,
        out_type=jax.ShapeDtypeStruct((B, D), jnp.float32),
        mesh=mesh,
        scratch_types=[
            pltpu.VMEM((b_per_w,), jnp.int32),
            pltpu.VMEM((NBUF, CH, D), jnp.float32),
            pltpu.SemaphoreType.DMA((NBUF,)),
            pltpu.SemaphoreType.DMA((NBUF,)),
        ],
    )
    def k(table_hbm, idx_hbm, out_hbm, idx_v, rows_v, gsem, psem):
        wid = lax.axis_index("s") * _NC + lax.axis_index("c")
        base = wid * b_per_w
        pltpu.sync_copy(idx_hbm.at[pl.ds(base, b_per_w)], idx_v)

        def gather(c, buf):
            return pltpu.async_copy(
                table_hbm.at[idx_v.at[pl.ds(c * CH, CH)]],
                rows_v.at[buf], gsem.at[buf])

        def put(c, buf):
            return pltpu.async_copy(
                rows_v.at[buf], out_hbm.at[pl.ds(base + c * CH, CH)],
                psem.at[buf])

        gets = [None] * NBUF
        puts = [None] * NBUF
        for c in range(min(NBUF, n_ch)):
            gets[c] = gather(c, c)
        for c in range(n_ch):
            buf = c % NBUF
            gets[buf].wait()
            puts[buf] = put(c, buf)
            nc = c + NBUF
            if nc < n_ch:
                puts[buf].wait()
                gets[buf] = gather(nc, buf)
        for c in range(max(0, n_ch - NBUF), n_ch):
            puts[c % NBUF].wait()

    return k(table, idx_flat)


def _time_mlp(t, W1, b1, W2, b2):
    D = W1.shape[0]
    H = W1.shape[1]
    half = D // 2
    n = t.shape[0]
    tp = jnp.pad(t, (0, 8 - n)).reshape(8, 1)
    freqs = jnp.exp(
        jnp.arange(half, dtype=jnp.float32) * (-math.log(10000.0) / half)
    ).reshape(1, half)

    KB = 1024
    nk = H // KB

    def body(t_ref, f_ref, W1_ref, b1_ref, W2_ref, b2_ref, o_ref):
        k = pl.program_id(0)
        args = (t_ref[...] * 1000.0) * f_ref[...]
        c = jnp.cos(args)
        s = jnp.sin(args)
        x = (c @ W1_ref[:half, :] + s @ W1_ref[half:, :]) + b1_ref[...]
        g = 0.5 * x * (1.0 + lax.erf(x * (1.0 / math.sqrt(2.0))))
        part = g @ W2_ref[...]

        @pl.when(k == 0)
        def _():
            o_ref[...] = part + b2_ref[...]

        @pl.when(k != 0)
        def _():
            o_ref[...] += part

    out = pl.pallas_call(
        body,
        grid=(nk,),
        in_specs=[
            pl.BlockSpec((8, 1), lambda k: (0, 0)),
            pl.BlockSpec((1, half), lambda k: (0, 0)),
            pl.BlockSpec((D, KB), lambda k: (0, k)),
            pl.BlockSpec((1, KB), lambda k: (0, k)),
            pl.BlockSpec((KB, D), lambda k: (k, 0)),
            pl.BlockSpec((1, D), lambda k: (0, 0)),
        ],
        out_specs=pl.BlockSpec((8, D), lambda k: (0, 0)),
        out_shape=jax.ShapeDtypeStruct((8, D), jnp.float32),
    )(tp, freqs, W1, b1.reshape(1, H), W2, b2.reshape(1, D))
    return out[:n]


def _fused_add_ln_half(g, pe, temb2, gamma, beta, scale, B_total,
                       batch_off, h=None):
    Bh, D = g.shape
    S = pe.shape[0]
    RB = 1024
    per_batch = S // RB
    n_batch = Bh // S

    def body(*refs):
        g_ref, pe_ref, te_ref, ga_ref, be_ref = refs[:5]
        o_ref = refs[-1]
        e = g_ref[...] * scale + pe_ref[...] + te_ref[0]
        mu = jnp.mean(e, axis=1, keepdims=True)
        ec = e - mu
        var = jnp.mean(ec * ec, axis=1, keepdims=True)
        o_ref[...] = ec * lax.rsqrt(var + 1e-5) * ga_ref[...] + be_ref[...]

    in_specs = [
        pl.BlockSpec((RB, D), lambda i, j: (j * per_batch + i, 0)),
        pl.BlockSpec((RB, D), lambda i, j: (i, 0)),
        pl.BlockSpec((1, 1, D), lambda i, j: (j + batch_off, 0, 0)),
        pl.BlockSpec((1, D), lambda i, j: (0, 0)),
        pl.BlockSpec((1, D), lambda i, j: (0, 0)),
    ]
    args = [g, pe, temb2.reshape(-1, 1, D),
            gamma.reshape(1, D), beta.reshape(1, D)]
    aliases = {}
    if h is not None:
        in_specs.append(pl.BlockSpec(memory_space=pl.ANY))
        args.append(h)
        aliases = {5: 0}
    return pl.pallas_call(
        body,
        grid=(per_batch, n_batch),
        in_specs=in_specs,
        out_specs=pl.BlockSpec(
            (RB, D), lambda i, j: ((j + batch_off) * per_batch + i, 0)),
        out_shape=jax.ShapeDtypeStruct((B_total, D), jnp.float32),
        input_output_aliases=aliases,
    )(*args)


def _make_pe(max_len, d_model):
    position = np.arange(max_len, dtype=np.float32)[:, None]
    div_term = np.exp(np.arange(0, d_model, 2, dtype=np.float32)
                      * (-np.log(10000.0) / d_model))
    pe = np.zeros((max_len, d_model), dtype=np.float32)
    pe[:, 0::2] = np.sin(position * div_term)
    pe[:, 1::2] = np.cos(position * div_term)
    return jnp.asarray(pe)


def kernel(x, t, table, W1, b1, W2, b2, gamma, beta):
    Bt, S = x.shape
    _, D = table.shape
    scale = math.sqrt(float(D))
    pe = _make_pe(S, D)

    idx = x.reshape(-1)
    B = idx.shape[0]
    Bh = B // 2
    nbh = Bt // 2

    g1 = _sc_gather(table, idx[:Bh])
    g2 = _sc_gather(table, idx[Bh:])
    temb2 = _time_mlp(t, W1, b1, W2, b2)
    h = _fused_add_ln_half(g1, pe, temb2, gamma, beta, scale, B, 0)
    out = _fused_add_ln_half(g2, pe, temb2, gamma, beta, scale, B, nbh, h=h)
    return out.reshape(Bt, S, D)


# speedup vs baseline: 1.0716x; 1.0716x over previous
import functools
import math

import jax
import jax.numpy as jnp
import numpy as np
from jax import lax
from jax.experimental import pallas as pl
from jax.experimental.pallas import tpu as pltpu
from jax.experimental.pallas import tpu_sc as plsc

_NC = 2
_NS = 16
_NW = _NC * _NS


def _sc_gather(table, idx_flat):
    B = idx_flat.shape[0]
    _, D = table.shape
    b_per_w = B // _NW
    CH = 32
    n_ch = b_per_w // CH
    mesh = plsc.VectorSubcoreMesh(core_axis_name="c", subcore_axis_name="s")

    NBUF = 3

    @functools.partial(
        pl.kernel,
        out_type=jax.ShapeDtypeStruct((B, D), jnp.float32),
        mesh=mesh,
        scratch_types=[
            pltpu.VMEM((b_per_w,), jnp.int32),
            pltpu.VMEM((NBUF, CH, D), jnp.float32),
            pltpu.SemaphoreType.DMA((NBUF,)),
            pltpu.SemaphoreType.DMA((NBUF,)),
        ],
    )
    def k(table_hbm, idx_hbm, out_hbm, idx_v, rows_v, gsem, psem):
        wid = lax.axis_index("s") * _NC + lax.axis_index("c")
        base = wid * b_per_w
        pltpu.sync_copy(idx_hbm.at[pl.ds(base, b_per_w)], idx_v)

        def gather(c, buf):
            return pltpu.async_copy(
                table_hbm.at[idx_v.at[pl.ds(c * CH, CH)]],
                rows_v.at[buf], gsem.at[buf])

        def put(c, buf):
            return pltpu.async_copy(
                rows_v.at[buf], out_hbm.at[pl.ds(base + c * CH, CH)],
                psem.at[buf])

        gets = [None] * NBUF
        puts = [None] * NBUF
        for c in range(min(NBUF, n_ch)):
            gets[c] = gather(c, c)
        for c in range(n_ch):
            buf = c % NBUF
            gets[buf].wait()
            puts[buf] = put(c, buf)
            nc = c + NBUF
            if nc < n_ch:
                puts[buf].wait()
                gets[buf] = gather(nc, buf)
        for c in range(max(0, n_ch - NBUF), n_ch):
            puts[c % NBUF].wait()

    return k(table, idx_flat)


def _time_mlp(t, W1, b1, W2, b2):
    D = W1.shape[0]
    H = W1.shape[1]
    half = D // 2
    n = t.shape[0]
    tp = jnp.pad(t, (0, 8 - n)).reshape(8, 1)
    freqs = jnp.exp(
        jnp.arange(half, dtype=jnp.float32) * (-math.log(10000.0) / half)
    ).reshape(1, half)

    KB = 1024
    nk = H // KB

    def body(t_ref, f_ref, W1_ref, b1_ref, W2_ref, b2_ref, o_ref):
        k = pl.program_id(0)
        args = (t_ref[...] * 1000.0) * f_ref[...]
        c = jnp.cos(args)
        s = jnp.sin(args)
        x = (c @ W1_ref[:half, :] + s @ W1_ref[half:, :]) + b1_ref[...]
        g = 0.5 * x * (1.0 + lax.erf(x * (1.0 / math.sqrt(2.0))))
        part = g @ W2_ref[...]

        @pl.when(k == 0)
        def _():
            o_ref[...] = part + b2_ref[...]

        @pl.when(k != 0)
        def _():
            o_ref[...] += part

    out = pl.pallas_call(
        body,
        grid=(nk,),
        in_specs=[
            pl.BlockSpec((8, 1), lambda k: (0, 0)),
            pl.BlockSpec((1, half), lambda k: (0, 0)),
            pl.BlockSpec((D, KB), lambda k: (0, k)),
            pl.BlockSpec((1, KB), lambda k: (0, k)),
            pl.BlockSpec((KB, D), lambda k: (k, 0)),
            pl.BlockSpec((1, D), lambda k: (0, 0)),
        ],
        out_specs=pl.BlockSpec((8, D), lambda k: (0, 0)),
        out_shape=jax.ShapeDtypeStruct((8, D), jnp.float32),
    )(tp, freqs, W1, b1.reshape(1, H), W2, b2.reshape(1, D))
    return out


def _fused_add_ln(g, pe, temb2, gamma, beta, scale):
    B, D = g.shape
    S = pe.shape[0]
    RB = 1024
    nb = B // RB
    per_batch = S // RB

    n_batch = nb // per_batch

    def body(g_ref, pe_ref, te_ref, ga_ref, be_ref, o_ref):
        e = g_ref[...] * scale + pe_ref[...].astype(jnp.float32) + te_ref[0]
        mu = jnp.mean(e, axis=1, keepdims=True)
        ec = e - mu
        var = jnp.mean(ec * ec, axis=1, keepdims=True)
        o_ref[...] = ec * lax.rsqrt(var + 1e-5) * ga_ref[...] + be_ref[...]

    return pl.pallas_call(
        body,
        grid=(per_batch, n_batch),
        in_specs=[
            pl.BlockSpec((RB, D), lambda i, j: (j * per_batch + i, 0)),
            pl.BlockSpec((RB, D), lambda i, j: (i, 0)),
            pl.BlockSpec((1, 1, D), lambda i, j: (j, 0, 0)),
            pl.BlockSpec((1, D), lambda i, j: (0, 0)),
            pl.BlockSpec((1, D), lambda i, j: (0, 0)),
        ],
        out_specs=pl.BlockSpec((RB, D), lambda i, j: (j * per_batch + i, 0)),
        out_shape=jax.ShapeDtypeStruct((B, D), jnp.float32),
    )(g, pe, temb2.reshape(-1, 1, D), gamma.reshape(1, D), beta.reshape(1, D))


def _make_pe(max_len, d_model):
    position = np.arange(max_len, dtype=np.float32)[:, None]
    div_term = np.exp(np.arange(0, d_model, 2, dtype=np.float32)
                      * (-np.log(10000.0) / d_model))
    pe = np.zeros((max_len, d_model), dtype=np.float32)
    pe[:, 0::2] = np.sin(position * div_term)
    pe[:, 1::2] = np.cos(position * div_term)
    return jnp.asarray(pe).astype(jnp.bfloat16)


def kernel(x, t, table, W1, b1, W2, b2, gamma, beta):
    Bt, S = x.shape
    _, D = table.shape
    scale = math.sqrt(float(D))
    pe = _make_pe(S, D)

    g = _sc_gather(table, x.reshape(-1))
    temb2 = _time_mlp(t, W1, b1, W2, b2)
    out = _fused_add_ln(g, pe, temb2, gamma, beta, scale)
    return out.reshape(Bt, S, D)


# speedup vs baseline: 1.0795x; 1.0074x over previous
import functools
import math

import jax
import jax.numpy as jnp
import numpy as np
from jax import lax
from jax.experimental import pallas as pl
from jax.experimental.pallas import tpu as pltpu
from jax.experimental.pallas import tpu_sc as plsc

_NC = 2
_NS = 16
_NW = _NC * _NS


def _sc_gather(table, idx_flat):
    B = idx_flat.shape[0]
    _, D = table.shape
    b_per_w = B // _NW
    CH = 32
    n_ch = b_per_w // CH
    mesh = plsc.VectorSubcoreMesh(core_axis_name="c", subcore_axis_name="s")

    NBUF = 3

    @functools.partial(
        pl.kernel,
        out_type=jax.ShapeDtypeStruct((B, D), jnp.float32),
        mesh=mesh,
        scratch_types=[
            pltpu.VMEM((b_per_w,), jnp.int32),
            pltpu.VMEM((NBUF, CH, D), jnp.float32),
            pltpu.SemaphoreType.DMA((NBUF,)),
            pltpu.SemaphoreType.DMA((NBUF,)),
        ],
    )
    def k(table_hbm, idx_hbm, out_hbm, idx_v, rows_v, gsem, psem):
        wid = lax.axis_index("s") * _NC + lax.axis_index("c")
        base = wid * b_per_w
        pltpu.sync_copy(idx_hbm.at[pl.ds(base, b_per_w)], idx_v)

        def gather(c, buf):
            return pltpu.async_copy(
                table_hbm.at[idx_v.at[pl.ds(c * CH, CH)]],
                rows_v.at[buf], gsem.at[buf])

        def put(c, buf):
            return pltpu.async_copy(
                rows_v.at[buf], out_hbm.at[pl.ds(base + c * CH, CH)],
                psem.at[buf])

        gets = [None] * NBUF
        puts = [None] * NBUF
        for c in range(min(NBUF, n_ch)):
            gets[c] = gather(c, c)
        for c in range(n_ch):
            buf = c % NBUF
            gets[buf].wait()
            puts[buf] = put(c, buf)
            nc = c + NBUF
            if nc < n_ch:
                puts[buf].wait()
                gets[buf] = gather(nc, buf)
        for c in range(max(0, n_ch - NBUF), n_ch):
            puts[c % NBUF].wait()

    return k(table, idx_flat)


def _time_mlp(t, W1, b1, W2, b2):
    D = W1.shape[0]
    H = W1.shape[1]
    half = D // 2
    n = t.shape[0]
    tp = jnp.pad(t, (0, 8 - n)).reshape(8, 1)
    freqs = jnp.exp(
        jnp.arange(half, dtype=jnp.float32) * (-math.log(10000.0) / half)
    ).reshape(1, half)

    KB = 1024
    nk = H // KB

    def body(t_ref, f_ref, W1_ref, b1_ref, W2_ref, b2_ref, o_ref):
        k = pl.program_id(0)
        args = (t_ref[...] * 1000.0) * f_ref[...]
        c = jnp.cos(args)
        s = jnp.sin(args)
        x = (c @ W1_ref[:half, :] + s @ W1_ref[half:, :]) + b1_ref[...]
        g = 0.5 * x * (1.0 + lax.erf(x * (1.0 / math.sqrt(2.0))))
        part = g @ W2_ref[...]

        @pl.when(k == 0)
        def _():
            o_ref[...] = part + b2_ref[...]

        @pl.when(k != 0)
        def _():
            o_ref[...] += part

    out = pl.pallas_call(
        body,
        grid=(nk,),
        in_specs=[
            pl.BlockSpec((8, 1), lambda k: (0, 0)),
            pl.BlockSpec((1, half), lambda k: (0, 0)),
            pl.BlockSpec((D, KB), lambda k: (0, k)),
            pl.BlockSpec((1, KB), lambda k: (0, k)),
            pl.BlockSpec((KB, D), lambda k: (k, 0)),
            pl.BlockSpec((1, D), lambda k: (0, 0)),
        ],
        out_specs=pl.BlockSpec((8, D), lambda k: (0, 0)),
        out_shape=jax.ShapeDtypeStruct((8, D), jnp.float32),
    )(tp, freqs, W1, b1.reshape(1, H), W2, b2.reshape(1, D))
    return out


def _fused_add_ln(g, pe, temb2, gamma, beta, scale):
    B, D = g.shape
    S = pe.shape[0]
    RB = 1024
    nb = B // RB
    per_batch = S // RB

    n_batch = nb // per_batch

    def body(g_ref, pe_ref, te_ref, ga_ref, be_ref, o_ref):
        e = g_ref[...] * scale + pe_ref[...].astype(jnp.float32) + te_ref[0]
        mu = jnp.mean(e, axis=1, keepdims=True)
        ec = e - mu
        var = jnp.mean(ec * ec, axis=1, keepdims=True)
        o_ref[...] = ec * lax.rsqrt(var + 1e-5) * ga_ref[...] + be_ref[...]

    return pl.pallas_call(
        body,
        grid=(per_batch, n_batch),
        in_specs=[
            pl.BlockSpec((RB, D), lambda i, j: (j * per_batch + i, 0)),
            pl.BlockSpec((RB, D), lambda i, j: (i, 0)),
            pl.BlockSpec((1, 1, D), lambda i, j: (j, 0, 0)),
            pl.BlockSpec((1, D), lambda i, j: (0, 0)),
            pl.BlockSpec((1, D), lambda i, j: (0, 0)),
        ],
        out_specs=pl.BlockSpec((RB, D), lambda i, j: (j * per_batch + i, 0)),
        out_shape=jax.ShapeDtypeStruct((B, D), jnp.float32),
        input_output_aliases={0: 0},
    )(g, pe, temb2.reshape(-1, 1, D), gamma.reshape(1, D), beta.reshape(1, D))


def _make_pe(max_len, d_model):
    position = np.arange(max_len, dtype=np.float32)[:, None]
    div_term = np.exp(np.arange(0, d_model, 2, dtype=np.float32)
                      * (-np.log(10000.0) / d_model))
    pe = np.zeros((max_len, d_model), dtype=np.float32)
    pe[:, 0::2] = np.sin(position * div_term)
    pe[:, 1::2] = np.cos(position * div_term)
    return jnp.asarray(pe).astype(jnp.bfloat16)


def kernel(x, t, table, W1, b1, W2, b2, gamma, beta):
    Bt, S = x.shape
    _, D = table.shape
    scale = math.sqrt(float(D))
    pe = _make_pe(S, D)

    g = _sc_gather(table, x.reshape(-1))
    temb2 = _time_mlp(t, W1, b1, W2, b2)
    out = _fused_add_ln(g, pe, temb2, gamma, beta, scale)
    return out.reshape(Bt, S, D)


# speedup vs baseline: 1.0810x; 1.0014x over previous
import functools
import math

import jax
import jax.numpy as jnp
import numpy as np
from jax import lax
from jax.experimental import pallas as pl
from jax.experimental.pallas import tpu as pltpu
from jax.experimental.pallas import tpu_sc as plsc

_NC = 2
_NS = 16
_NW = _NC * _NS


def _sc_gather(table, idx_flat):
    B = idx_flat.shape[0]
    _, D = table.shape
    b_per_w = B // _NW
    CH = 32
    n_ch = b_per_w // CH
    mesh = plsc.VectorSubcoreMesh(core_axis_name="c", subcore_axis_name="s")

    NBUF = 3

    @functools.partial(
        pl.kernel,
        out_type=jax.ShapeDtypeStruct((B, D), jnp.float32),
        mesh=mesh,
        scratch_types=[
            pltpu.VMEM((b_per_w,), jnp.int32),
            pltpu.VMEM((NBUF, CH, D), jnp.float32),
            pltpu.SemaphoreType.DMA((NBUF,)),
            pltpu.SemaphoreType.DMA((NBUF,)),
        ],
    )
    def k(table_hbm, idx_hbm, out_hbm, idx_v, rows_v, gsem, psem):
        wid = lax.axis_index("s") * _NC + lax.axis_index("c")
        base = wid * b_per_w
        pltpu.sync_copy(idx_hbm.at[pl.ds(base, b_per_w)], idx_v)

        def gather(c, buf):
            return pltpu.async_copy(
                table_hbm.at[idx_v.at[pl.ds(c * CH, CH)]],
                rows_v.at[buf], gsem.at[buf])

        def put(c, buf):
            return pltpu.async_copy(
                rows_v.at[buf], out_hbm.at[pl.ds(base + c * CH, CH)],
                psem.at[buf])

        gets = [None] * NBUF
        puts = [None] * NBUF
        for c in range(min(NBUF, n_ch)):
            gets[c] = gather(c, c)
        for c in range(n_ch):
            buf = c % NBUF
            gets[buf].wait()
            puts[buf] = put(c, buf)
            nc = c + NBUF
            if nc < n_ch:
                puts[buf].wait()
                gets[buf] = gather(nc, buf)
        for c in range(max(0, n_ch - NBUF), n_ch):
            puts[c % NBUF].wait()

    return k(table, idx_flat)


def _time_mlp(t, W1, b1, W2, b2):
    D = W1.shape[0]
    H = W1.shape[1]
    half = D // 2
    n = t.shape[0]
    tp = jnp.pad(t, (0, 8 - n)).reshape(8, 1)
    freqs = jnp.exp(
        jnp.arange(half, dtype=jnp.float32) * (-math.log(10000.0) / half)
    ).reshape(1, half)

    KB = 1024
    nk = H // KB

    def body(t_ref, f_ref, W1_ref, b1_ref, W2_ref, b2_ref, o_ref):
        k = pl.program_id(0)
        args = (t_ref[...] * 1000.0) * f_ref[...]
        c = jnp.cos(args)
        s = jnp.sin(args)
        x = (c @ W1_ref[:half, :] + s @ W1_ref[half:, :]) + b1_ref[...]
        g = 0.5 * x * (1.0 + lax.erf(x * (1.0 / math.sqrt(2.0))))
        part = g @ W2_ref[...]

        @pl.when(k == 0)
        def _():
            o_ref[...] = part + b2_ref[...]

        @pl.when(k != 0)
        def _():
            o_ref[...] += part

    out = pl.pallas_call(
        body,
        grid=(nk,),
        in_specs=[
            pl.BlockSpec((8, 1), lambda k: (0, 0)),
            pl.BlockSpec((1, half), lambda k: (0, 0)),
            pl.BlockSpec((D, KB), lambda k: (0, k)),
            pl.BlockSpec((1, KB), lambda k: (0, k)),
            pl.BlockSpec((KB, D), lambda k: (k, 0)),
            pl.BlockSpec((1, D), lambda k: (0, 0)),
        ],
        out_specs=pl.BlockSpec((8, D), lambda k: (0, 0)),
        out_shape=jax.ShapeDtypeStruct((8, D), jnp.float32),
    )(tp, freqs, W1, b1.reshape(1, H), W2, b2.reshape(1, D))
    return out


def _fused_add_ln(g, pe, temb2, gamma, beta, scale):
    B, D = g.shape
    S = pe.shape[0]
    RB = 1024
    nb = B // RB
    per_batch = S // RB

    n_batch = nb // per_batch

    def body(g_ref, pe_ref, te_ref, ga_ref, be_ref, o_ref):
        e = g_ref[...] * scale + pe_ref[...].astype(jnp.float32) + te_ref[0]
        mu = jnp.mean(e, axis=1, keepdims=True)
        ec = e - mu
        var = jnp.mean(ec * ec, axis=1, keepdims=True)
        o_ref[...] = ec * lax.rsqrt(var + 1e-5) * ga_ref[...] + be_ref[...]

    return pl.pallas_call(
        body,
        grid=(per_batch, n_batch),
        in_specs=[
            pl.BlockSpec((RB, D), lambda i, j: (j * per_batch + i, 0)),
            pl.BlockSpec((RB, D), lambda i, j: (i, 0)),
            pl.BlockSpec((1, 1, D), lambda i, j: (j, 0, 0)),
            pl.BlockSpec((1, D), lambda i, j: (0, 0)),
            pl.BlockSpec((1, D), lambda i, j: (0, 0)),
        ],
        out_specs=pl.BlockSpec((RB, D), lambda i, j: (j * per_batch + i, 0)),
        out_shape=jax.ShapeDtypeStruct((B, D), jnp.float32),
    )(g, pe, temb2.reshape(-1, 1, D), gamma.reshape(1, D), beta.reshape(1, D))


def _make_pe(max_len, d_model):
    position = np.arange(max_len, dtype=np.float32)[:, None]
    div_term = np.exp(np.arange(0, d_model, 2, dtype=np.float32)
                      * (-np.log(10000.0) / d_model))
    pe = np.zeros((max_len, d_model), dtype=np.float32)
    pe[:, 0::2] = np.sin(position * div_term)
    pe[:, 1::2] = np.cos(position * div_term)
    return jnp.asarray(pe).astype(jnp.bfloat16)


def kernel(x, t, table, W1, b1, W2, b2, gamma, beta):
    Bt, S = x.shape
    _, D = table.shape
    scale = math.sqrt(float(D))
    pe = _make_pe(S, D)

    g = _sc_gather(table, x.reshape(-1))
    temb2 = _time_mlp(t, W1, b1, W2, b2)
    out = _fused_add_ln(g, pe, temb2, gamma, beta, scale)
    return out.reshape(Bt, S, D)
